# Initial kernel scaffold; baseline (speedup 1.0000x reference)
import jax
import jax.numpy as jnp
from jax.experimental import pallas as pl


def kernel(t_id, table, W, b):
    raise NotImplementedError("write your pallas kernel here")


# speedup vs baseline: 1.0177x; 1.0177x over previous
import functools

import jax
import jax.numpy as jnp
from jax import lax
from jax.experimental import pallas as pl
from jax.experimental.pallas import tpu as pltpu
from jax.experimental.pallas import tpu_sc as plsc


@functools.lru_cache(maxsize=None)
def _make_sc_gather(V, D, N):
    info = plsc.get_sparse_core_info()
    NC, NS = info.num_cores, info.num_subcores
    NW = NC * NS
    n_per_w = N // NW
    CH = 512
    IR = CH // 128
    n_chunks = n_per_w // CH
    ir_per_w = n_per_w // 128
    mesh = plsc.VectorSubcoreMesh(core_axis_name="c", subcore_axis_name="s")

    @functools.partial(
        pl.kernel,
        mesh=mesh,
        out_type=jax.ShapeDtypeStruct((N, D), jnp.float32),
        scratch_types=[
            pltpu.VMEM((IR, 128), jnp.int32),
            pltpu.VMEM((CH, D), jnp.float32),
            pltpu.SemaphoreType.DMA,
        ],
        compiler_params=pltpu.CompilerParams(use_tc_tiling_on_sc=False),
    )
    def gather(table_hbm, idx_hbm, out_hbm, idx_v, rows_v, sem):
        wid = lax.axis_index("s") * NC + lax.axis_index("c")

        def chunk(k, carry):
            irow = wid * ir_per_w + k * IR
            rbase = wid * n_per_w + k * CH
            pltpu.sync_copy(idx_hbm.at[pl.ds(irow, IR)], idx_v)
            cps = [
                pltpu.async_copy(
                    table_hbm.at[idx_v.at[j]],
                    rows_v.at[pl.ds(j * 128, 128)],
                    sem,
                )
                for j in range(IR)
            ]
            for cp in cps:
                cp.wait()
            pltpu.sync_copy(rows_v, out_hbm.at[pl.ds(rbase, CH)])
            return carry

        lax.fori_loop(0, n_chunks, chunk, 0)

    return gather


def _tc_project(emb, Wt, b2):
    N, D = emb.shape
    O = Wt.shape[1]
    R = 2048

    def mm(x_ref, wt_ref, b_ref, o_ref):
        o_ref[...] = (
            jnp.dot(x_ref[...], wt_ref[...], preferred_element_type=jnp.float32)
            + b_ref[...]
        )

    return pl.pallas_call(
        mm,
        grid=(N // R,),
        in_specs=[
            pl.BlockSpec((R, D), lambda i: (i, 0)),
            pl.BlockSpec((D, O), lambda i: (0, 0)),
            pl.BlockSpec((1, O), lambda i: (0, 0)),
        ],
        out_specs=pl.BlockSpec((R, O), lambda i: (i, 0)),
        out_shape=jax.ShapeDtypeStruct((N, O), jnp.float32),
    )(emb, Wt, b2)


def kernel(t_id, table, W, b):
    B, L = t_id.shape
    V, D = table.shape
    O = W.shape[0]
    N = B * L
    idx = t_id.astype(jnp.int32).reshape(N // 128, 128)
    emb = _make_sc_gather(V, D, N)(table, idx)
    out = _tc_project(emb, W.T, b.reshape(1, O))
    return out.reshape(B, L, O)


# speedup vs baseline: 3.3712x; 3.3125x over previous
import functools

import jax
import jax.numpy as jnp
from jax import lax
from jax.experimental import pallas as pl
from jax.experimental.pallas import tpu as pltpu
from jax.experimental.pallas import tpu_sc as plsc


def _project(tT, Wt, b2):
    D, V = tT.shape
    O = Wt.shape[1]
    R = 4096

    def body(x_ref, wt_ref, b_ref, o_ref):
        o_ref[...] = (
            lax.dot_general(
                x_ref[...], wt_ref[...], (((0,), (0,)), ((), ())),
                preferred_element_type=jnp.float32,
            )
            + b_ref[...]
        )

    return pl.pallas_call(
        body,
        grid=(pl.cdiv(V, R),),
        in_specs=[
            pl.BlockSpec((D, R), lambda i: (0, i)),
            pl.BlockSpec((D, O), lambda i: (0, 0)),
            pl.BlockSpec((1, O), lambda i: (0, 0)),
        ],
        out_specs=pl.BlockSpec((R, O), lambda i: (i, 0)),
        out_shape=jax.ShapeDtypeStruct((V, O), jnp.float32),
    )(tT, Wt, b2)


@functools.lru_cache(maxsize=None)
def _make_sc_gather(V, D, N):
    info = plsc.get_sparse_core_info()
    NC, NS = info.num_cores, info.num_subcores
    NW = NC * NS
    n_per_w = N // NW
    CH = 512
    IR = CH // 128
    n_chunks = n_per_w // CH
    ir_per_w = n_per_w // 128
    mesh = plsc.VectorSubcoreMesh(core_axis_name="c", subcore_axis_name="s")

    @functools.partial(
        pl.kernel,
        mesh=mesh,
        out_type=jax.ShapeDtypeStruct((N, D), jnp.float32),
        scratch_types=[
            pltpu.VMEM((IR, 128), jnp.int32),
            pltpu.VMEM((CH, D), jnp.float32),
            pltpu.SemaphoreType.DMA,
        ],
        compiler_params=pltpu.CompilerParams(use_tc_tiling_on_sc=False),
    )
    def gather(table_hbm, idx_hbm, out_hbm, idx_v, rows_v, sem):
        wid = lax.axis_index("s") * NC + lax.axis_index("c")

        def chunk(k, carry):
            irow = wid * ir_per_w + k * IR
            rbase = wid * n_per_w + k * CH
            pltpu.sync_copy(idx_hbm.at[pl.ds(irow, IR)], idx_v)
            cps = [
                pltpu.async_copy(
                    table_hbm.at[idx_v.at[j]],
                    rows_v.at[pl.ds(j * 128, 128)],
                    sem,
                )
                for j in range(IR)
            ]
            for cp in cps:
                cp.wait()
            pltpu.sync_copy(rows_v, out_hbm.at[pl.ds(rbase, CH)])
            return carry

        lax.fori_loop(0, n_chunks, chunk, 0)

    return gather


def kernel(t_id, table, W, b):
    B, L = t_id.shape
    V, D = table.shape
    O = W.shape[0]
    N = B * L
    P = _project(table.T, W.T, b.reshape(1, O))
    idx = t_id.T.astype(jnp.int32).reshape(N // 128, 128)
    outT = _make_sc_gather(V, O, N)(P, idx)
    return outT.reshape(L, B, O).transpose(1, 0, 2)
